# baseline (speedup 1.0000x reference)
import jax
import jax.numpy as jnp
from jax import lax
from jax.experimental import pallas as pl
from jax.experimental.pallas import tpu as pltpu
from jax.experimental.pallas import tpu_sc as plsc

B, S, D, E = 4, 8192, 2048, 64
NC, NS = 2, 16
NW = NC * NS
JW = NW // B
CPW = D // JW
NV = CPW // 16
RCHUNK = 64
NBUF = 4
NCHUNKS = S // RCHUNK
NGROUPS = NCHUNKS // NBUF


def _sc_reduce_body(h_hbm, out_hbm, b0, b1, b2, b3, accv, s0, s1, s2, s3):
    bufs = [b0, b1, b2, b3]
    sems = [s0, s1, s2, s3]
    c = lax.axis_index("c")
    s = lax.axis_index("s")
    w = s * NC + c
    b = w // JW
    col0 = (w % JW) * CPW

    def src(ci):
        return h_hbm.at[b, pl.ds(ci * RCHUNK, RCHUNK), pl.ds(col0, CPW)]

    for k in range(NBUF):
        pltpu.make_async_copy(src(k), bufs[k], sems[k]).start()

    def consume(k, ci, acc):
        pltpu.make_async_copy(src(ci), bufs[k], sems[k]).wait()

        def rows(r, a):
            return tuple(a[q] + bufs[k][r, pl.ds(q * 16, 16)] for q in range(NV))

        return lax.fori_loop(0, RCHUNK, rows, acc, unroll=2)

    def group(g, acc):
        for k in range(NBUF):
            ci = g * NBUF + k
            acc = consume(k, ci, acc)
            pltpu.make_async_copy(src(ci + NBUF), bufs[k], sems[k]).start()
        return acc

    acc = tuple(jnp.zeros((16,), jnp.float32) for _ in range(NV))
    acc = lax.fori_loop(0, NGROUPS - 1, group, acc)
    for k in range(NBUF):
        acc = consume(k, (NGROUPS - 1) * NBUF + k, acc)

    for q in range(NV):
        accv[pl.ds(q * 16, 16)] = acc[q]
    pltpu.sync_copy(accv, out_hbm.at[b, pl.ds(col0, CPW)])


_sc_reduce = pl.kernel(
    _sc_reduce_body,
    out_type=jax.ShapeDtypeStruct((B, D), jnp.float32),
    mesh=plsc.VectorSubcoreMesh(core_axis_name="c", subcore_axis_name="s"),
    scratch_types=(
        [pltpu.VMEM((RCHUNK, CPW), jnp.float32) for _ in range(NBUF)]
        + [pltpu.VMEM((CPW,), jnp.float32)]
        + [pltpu.SemaphoreType.DMA for _ in range(NBUF)]
    ),
)


def _finish_body(sum_ref, w_ref, o_ref):
    pooled = sum_ref[...] * (1.0 / S)
    logits = lax.dot_general(
        pooled, w_ref[...],
        dimension_numbers=(((1,), (1,)), ((), ())),
        preferred_element_type=jnp.float32,
    )
    m = jnp.max(logits, axis=-1, keepdims=True)
    e = jnp.exp(logits - m)
    o_ref[...] = e / jnp.sum(e, axis=-1, keepdims=True)


def _finish(pooled_sum, W):
    return pl.pallas_call(
        _finish_body,
        out_shape=jax.ShapeDtypeStruct((B, E), jnp.float32),
    )(pooled_sum, W)


def kernel(hidden_states, W):
    return _finish(_sc_reduce(hidden_states), W)


# speedup vs baseline: 1.0561x; 1.0561x over previous
import jax
import jax.numpy as jnp
from jax import lax
from jax.experimental import pallas as pl
from jax.experimental.pallas import tpu as pltpu
from jax.experimental.pallas import tpu_sc as plsc

B, S, D, E = 4, 8192, 2048, 64
NC, NS = 2, 16
NW = NC * NS
JW = NW // B
CPW = D // JW
NV = CPW // 16
RCHUNK = 64
NBUF = 4
SSC = 4096
NCHUNKS = SSC // RCHUNK
NGROUPS = NCHUNKS // NBUF
TCHUNK = 512
TGRID = (S - SSC) // TCHUNK


def _sc_reduce_body(h_hbm, out_hbm, b0, b1, b2, b3, accv, s0, s1, s2, s3):
    bufs = [b0, b1, b2, b3]
    sems = [s0, s1, s2, s3]
    c = lax.axis_index("c")
    s = lax.axis_index("s")
    w = s * NC + c
    b = w // JW
    col0 = (w % JW) * CPW

    def src(ci):
        return h_hbm.at[b, pl.ds(ci * RCHUNK, RCHUNK), pl.ds(col0, CPW)]

    for k in range(NBUF):
        pltpu.make_async_copy(src(k), bufs[k], sems[k]).start()

    def consume(k, ci, acc):
        pltpu.make_async_copy(src(ci), bufs[k], sems[k]).wait()

        def rows(r, a):
            return tuple(a[q] + bufs[k][r, pl.ds(q * 16, 16)] for q in range(NV))

        return lax.fori_loop(0, RCHUNK, rows, acc, unroll=2)

    def group(g, acc):
        for k in range(NBUF):
            ci = g * NBUF + k
            acc = consume(k, ci, acc)
            pltpu.make_async_copy(src(ci + NBUF), bufs[k], sems[k]).start()
        return acc

    acc = tuple(jnp.zeros((16,), jnp.float32) for _ in range(NV))
    acc = lax.fori_loop(0, NGROUPS - 1, group, acc)
    for k in range(NBUF):
        acc = consume(k, (NGROUPS - 1) * NBUF + k, acc)

    for q in range(NV):
        accv[pl.ds(q * 16, 16)] = acc[q]
    pltpu.sync_copy(accv, out_hbm.at[b, pl.ds(col0, CPW)])


_sc_reduce = pl.kernel(
    _sc_reduce_body,
    out_type=jax.ShapeDtypeStruct((B, D), jnp.float32),
    mesh=plsc.VectorSubcoreMesh(core_axis_name="c", subcore_axis_name="s"),
    scratch_types=(
        [pltpu.VMEM((RCHUNK, CPW), jnp.float32) for _ in range(NBUF)]
        + [pltpu.VMEM((CPW,), jnp.float32)]
        + [pltpu.SemaphoreType.DMA for _ in range(NBUF)]
    ),
)


def _tc_reduce_body(h_ref, o_ref):
    i = pl.program_id(0)

    @pl.when(i == 0)
    def _init():
        o_ref[...] = jnp.zeros_like(o_ref)

    o_ref[...] += jnp.sum(h_ref[...], axis=1)


def _tc_reduce(hidden_states):
    return pl.pallas_call(
        _tc_reduce_body,
        grid=(TGRID,),
        in_specs=[pl.BlockSpec((B, TCHUNK, D), lambda i: (0, SSC // TCHUNK + i, 0))],
        out_specs=pl.BlockSpec((B, D), lambda i: (0, 0)),
        out_shape=jax.ShapeDtypeStruct((B, D), jnp.float32),
    )(hidden_states)


def _finish_body(a_ref, b_ref, w_ref, o_ref):
    pooled = (a_ref[...] + b_ref[...]) * (1.0 / S)
    logits = lax.dot_general(
        pooled, w_ref[...],
        dimension_numbers=(((1,), (1,)), ((), ())),
        preferred_element_type=jnp.float32,
    )
    m = jnp.max(logits, axis=-1, keepdims=True)
    e = jnp.exp(logits - m)
    o_ref[...] = e / jnp.sum(e, axis=-1, keepdims=True)


def _finish(sc_sum, tc_sum, W):
    return pl.pallas_call(
        _finish_body,
        out_shape=jax.ShapeDtypeStruct((B, E), jnp.float32),
    )(sc_sum, tc_sum, W)


def kernel(hidden_states, W):
    sc_sum = _sc_reduce(hidden_states)
    tc_sum = _tc_reduce(hidden_states)
    return _finish(sc_sum, tc_sum, W)
